# Initial kernel scaffold; baseline (speedup 1.0000x reference)
import jax
import jax.numpy as jnp
from jax.experimental import pallas as pl


def kernel(tokenized_sentences, embedding_table):
    raise NotImplementedError("write your pallas kernel here")


# speedup vs baseline: 1.7418x; 1.7418x over previous
import functools

import numpy as np
import jax
import jax.numpy as jnp
from jax import lax
from jax.experimental import pallas as pl
from jax.experimental.pallas import tpu as pltpu
from jax.experimental.pallas import tpu_sc as plsc

_EMB = 64
_SEQ = 50
_BATCH = 16384
_NPARAM = 10000
_B = _BATCH * _SEQ
_NW = 32
_PER_W = _B // _NW
_W = 800
_NCHUNK = _PER_W // _W
_SUB = 80
_NSUB = _W // _SUB
_LANES = 16


def _pos_table():
    pos = np.arange(_SEQ, dtype=np.float64)[:, None]
    dim = np.arange(_EMB // 2, dtype=np.float64)[None, :]
    theta = pos / (_NPARAM ** (2.0 * dim / _EMB))
    pe = np.zeros((_SEQ, _EMB), dtype=np.float64)
    pe[:, 0::2] = np.sin(theta)
    pe[:, 1::2] = np.cos(theta)
    return pe.astype(np.float32)


_POS = _pos_table()


def _sc_embed(idx_flat, table, pos):
    mesh = plsc.VectorSubcoreMesh(
        core_axis_name="c", subcore_axis_name="s", num_cores=2, num_subcores=16
    )

    @functools.partial(
        pl.kernel,
        out_type=jax.ShapeDtypeStruct((_B, _EMB), jnp.float32),
        mesh=mesh,
        scratch_types=[
            pltpu.VMEM((_W,), jnp.int32),
            pltpu.VMEM((_W, _EMB), jnp.float32),
            pltpu.VMEM((_SEQ, _EMB), jnp.float32),
            pltpu.SemaphoreType.DMA,
        ],
        compiler_params=pltpu.CompilerParams(use_tc_tiling_on_sc=False),
    )
    def k(idx_hbm, tab_hbm, pos_hbm, out_hbm, idx_v, rows_v, pos_v, gsem):
        wid = lax.axis_index("s") * 2 + lax.axis_index("c")
        base0 = wid * _PER_W
        pltpu.sync_copy(pos_hbm, pos_v)

        @pl.loop(0, _NCHUNK)
        def _chunk_loop(chunk):
            base = base0 + chunk * _W
            pltpu.sync_copy(idx_hbm.at[pl.ds(base, _W)], idx_v)
            copies = [
                pltpu.async_copy(
                    tab_hbm.at[idx_v.at[pl.ds(j * _SUB, _SUB)]],
                    rows_v.at[pl.ds(j * _SUB, _SUB)],
                    gsem,
                )
                for j in range(_NSUB)
            ]
            for cp in copies:
                cp.wait()

            @pl.loop(0, _SEQ)
            def _row_loop(s):
                for c in range(_EMB // _LANES):
                    pv = pos_v[s, pl.ds(c * _LANES, _LANES)]

                    @pl.loop(0, _W, step=_SEQ)
                    def _sent_loop(b, pv=pv, s=s, c=c):
                        r = b + s
                        sl = pl.ds(c * _LANES, _LANES)
                        rows_v[r, sl] = rows_v[r, sl] * 8.0 + pv

            pltpu.sync_copy(rows_v, out_hbm.at[pl.ds(base, _W)])

    return k(idx_flat, table, pos)


def _tc_mask(tok):
    def body(t_ref, m_ref):
        m_ref[...] = t_ref[...] != 0

    return pl.pallas_call(
        body,
        out_shape=jax.ShapeDtypeStruct((_BATCH, _SEQ), jnp.bool_),
        grid=(16,),
        in_specs=[pl.BlockSpec((_BATCH // 16, _SEQ), lambda i: (i, 0))],
        out_specs=pl.BlockSpec((_BATCH // 16, _SEQ), lambda i: (i, 0)),
    )(tok)


def kernel(tokenized_sentences, embedding_table):
    tok = tokenized_sentences.astype(jnp.int32)
    idx_flat = tok.reshape(_B)
    enc = _sc_embed(idx_flat, embedding_table, jnp.asarray(_POS))
    mask = _tc_mask(tok)
    return enc.reshape(_BATCH, _SEQ, _EMB), mask
